# baseline (speedup 1.0000x reference)
import functools

import jax
import jax.numpy as jnp
from jax import lax
from jax.experimental import pallas as pl
from jax.experimental.pallas import tpu as pltpu
from jax.experimental.pallas import tpu_sc as plsc

EPS = 1e-5
ROWB = 1000
NC = 2
NS = 16
CH = 128


def _mesh():
    return plsc.VectorSubcoreMesh(
        core_axis_name="c", subcore_axis_name="s", num_cores=NC, num_subcores=NS
    )


def _zero_fill(buf, rows, width):
    per_row = width // 16

    def body(k, _):
        buf[k // per_row, pl.ds((k % per_row) * 16, 16)] = jnp.zeros((16,), jnp.float32)
        return 0

    lax.fori_loop(0, rows * per_row, body, 0)


def _sc_degree(dst2d, n):
    r = dst2d.shape[0]
    rw = r // (NC * NS)
    rpt = n // NS
    zbr = rpt // 5

    @functools.partial(
        pl.kernel,
        out_type=jax.ShapeDtypeStruct((NC, n, 16), jnp.float32),
        mesh=_mesh(),
        scratch_types=[
            pltpu.VMEM((rw, CH), jnp.int32),
            pltpu.VMEM((CH, 16), jnp.float32),
            pltpu.VMEM((zbr, 16), jnp.float32),
            pltpu.VMEM_SHARED((n, 16), jnp.float32),
        ],
    )
    def k(dst_hbm, out_hbm, dstv, ones_v, zbuf, acc):
        c = lax.axis_index("c")
        s = lax.axis_index("s")
        wid = s * NC + c

        def init_body(i, _):
            ones_v[i, :] = jnp.ones((16,), jnp.float32)
            return 0

        lax.fori_loop(0, CH, init_body, 0)
        _zero_fill(zbuf, zbr, 16)
        for t in range(rpt // zbr):
            pltpu.sync_copy(zbuf, acc.at[pl.ds(s * rpt + t * zbr, zbr)])
        pltpu.sync_copy(dst_hbm.at[pl.ds(wid * rw, rw)], dstv)
        plsc.subcore_barrier()

        def body(j, _):
            pltpu.sync_copy(
                ones_v,
                acc.at[plsc.Indices(dstv.at[j], ignored_value=-1)],
                add=True,
            )
            return 0

        lax.fori_loop(0, rw, body, 0)
        plsc.subcore_barrier()
        pltpu.sync_copy(acc.at[pl.ds(s * rpt, rpt)], out_hbm.at[c, pl.ds(s * rpt, rpt)])

    return k(dst2d)


def _sc_agg_cols(src2d, dst2d, u, n, passes=2):
    r = src2d.shape[0]
    rs_ = r // NS
    w = u.shape[1]
    hw = w // NC
    half = n // passes
    rpt = half // NS
    zbr = rpt // 5

    @functools.partial(
        pl.kernel,
        out_type=jax.ShapeDtypeStruct((n, w), jnp.float32),
        mesh=_mesh(),
        scratch_types=[
            pltpu.VMEM((rs_, CH), jnp.int32),
            pltpu.VMEM((rs_, CH), jnp.int32),
            pltpu.VMEM((1, CH), jnp.int32),
            pltpu.VMEM((1, CH), jnp.int32),
            pltpu.VMEM((CH, hw), jnp.float32),
            pltpu.VMEM((zbr, hw), jnp.float32),
            pltpu.VMEM_SHARED((half, hw), jnp.float32),
            pltpu.SemaphoreType.DMA,
        ],
    )
    def k(src_hbm, dst_hbm, u_hbm, z_hbm, srcv, dstv, msrcv, mdstv, gbuf, zbuf,
          acc, sem):
        c = lax.axis_index("c")
        s = lax.axis_index("s")
        cbase = c * hw
        _zero_fill(zbuf, zbr, hw)
        pltpu.sync_copy(src_hbm.at[pl.ds(s * rs_, rs_)], srcv)
        pltpu.sync_copy(dst_hbm.at[pl.ds(s * rs_, rs_)], dstv)

        for p in range(passes):
            base = p * half
            for t in range(rpt // zbr):
                pltpu.sync_copy(zbuf, acc.at[pl.ds(s * rpt + t * zbr, zbr)])
            plsc.subcore_barrier()

            def body(j, _):
                for kk in range(CH // 16):
                    sl = pl.ds(kk * 16, 16)
                    s16 = srcv[j, sl]
                    d16 = dstv[j, sl]
                    inr = (d16 >= base) & (d16 < base + half)
                    msrcv[0, sl] = jnp.where(inr, s16, 0)
                    mdstv[0, sl] = jnp.where(inr, d16 - base, -1)
                pltpu.async_copy(
                    u_hbm.at[msrcv.at[0], pl.ds(cbase, hw)],
                    gbuf,
                    sem,
                ).wait()
                pltpu.sync_copy(
                    gbuf,
                    acc.at[plsc.Indices(mdstv.at[0], ignored_value=-1)],
                    add=True,
                )
                return 0

            lax.fori_loop(0, rs_, body, 0)
            plsc.subcore_barrier()
            pltpu.sync_copy(
                acc.at[pl.ds(s * rpt, rpt)],
                z_hbm.at[pl.ds(base + s * rpt, rpt), pl.ds(cbase, hw)],
            )
            if p + 1 < passes:
                plsc.subcore_barrier()

    return k(src2d, dst2d, u)


def _tc_lin1(x, w1, b1, degp):
    n, d = x.shape
    h = w1.shape[1]
    nb = n // ROWB

    def body(x_ref, w_ref, b_ref, deg_ref, u_ref, dinv_ref):
        deg = deg_ref[0, :, 0] + deg_ref[1, :, 0] + 1.0
        dv = lax.rsqrt(deg)
        dinv_ref[...] = dv[:, None]
        hh = jnp.dot(x_ref[...], w_ref[...], preferred_element_type=jnp.float32)
        u_ref[...] = (hh + b_ref[...]) * dv[:, None]

    return pl.pallas_call(
        body,
        grid=(nb,),
        in_specs=[
            pl.BlockSpec((ROWB, d), lambda i: (i, 0)),
            pl.BlockSpec((d, h), lambda i: (0, 0)),
            pl.BlockSpec((1, h), lambda i: (0, 0)),
            pl.BlockSpec((NC, ROWB, 16), lambda i: (0, i, 0)),
        ],
        out_specs=[
            pl.BlockSpec((ROWB, h), lambda i: (i, 0)),
            pl.BlockSpec((ROWB, 1), lambda i: (i, 0)),
        ],
        out_shape=[
            jax.ShapeDtypeStruct((n, h), jnp.float32),
            jax.ShapeDtypeStruct((n, 1), jnp.float32),
        ],
    )(x, w1, b1.reshape(1, h), degp)


def _tc_stats(z, u, dinv):
    n, h = u.shape
    nb = n // ROWB

    def body(z_ref, u_ref, dv_ref, s_ref):
        i = pl.program_id(0)
        a = (z_ref[...] + u_ref[...]) * dv_ref[...]

        @pl.when(i == 0)
        def _():
            s_ref[...] = jnp.zeros_like(s_ref)

        s_ref[0:1, :] += jnp.sum(a, axis=0, keepdims=True)
        s_ref[1:2, :] += jnp.sum(a * a, axis=0, keepdims=True)

    return pl.pallas_call(
        body,
        grid=(nb,),
        in_specs=[
            pl.BlockSpec((ROWB, h), lambda i: (i, 0)),
            pl.BlockSpec((ROWB, h), lambda i: (i, 0)),
            pl.BlockSpec((ROWB, 1), lambda i: (i, 0)),
        ],
        out_specs=pl.BlockSpec((8, h), lambda i: (0, 0)),
        out_shape=jax.ShapeDtypeStruct((8, h), jnp.float32),
    )(z, u, dinv)


def _tc_layer(z, u, dinv, st, g, be, w, b, n_total):
    n, h = u.shape
    ho = w.shape[1]
    nb = n // ROWB

    def body(z_ref, u_ref, dv_ref, st_ref, g_ref, be_ref, w_ref, b_ref, o_ref):
        a = (z_ref[...] + u_ref[...]) * dv_ref[...]
        mu = st_ref[0:1, :] * (1.0 / n_total)
        ex2 = st_ref[1:2, :] * (1.0 / n_total)
        var = ex2 - mu * mu
        sc = g_ref[...] * lax.rsqrt(var + EPS)
        tt = be_ref[...] - mu * sc
        hn = jnp.maximum(a * sc + tt, 0.0)
        hh = jnp.dot(hn, w_ref[...], preferred_element_type=jnp.float32)
        o_ref[...] = (hh + b_ref[...]) * dv_ref[...]

    return pl.pallas_call(
        body,
        grid=(nb,),
        in_specs=[
            pl.BlockSpec((ROWB, h), lambda i: (i, 0)),
            pl.BlockSpec((ROWB, h), lambda i: (i, 0)),
            pl.BlockSpec((ROWB, 1), lambda i: (i, 0)),
            pl.BlockSpec((8, h), lambda i: (0, 0)),
            pl.BlockSpec((1, h), lambda i: (0, 0)),
            pl.BlockSpec((1, h), lambda i: (0, 0)),
            pl.BlockSpec((h, ho), lambda i: (0, 0)),
            pl.BlockSpec((1, ho), lambda i: (0, 0)),
        ],
        out_specs=pl.BlockSpec((ROWB, ho), lambda i: (i, 0)),
        out_shape=jax.ShapeDtypeStruct((n, ho), jnp.float32),
    )(z, u, dinv, st, g.reshape(1, h), be.reshape(1, h), w, b.reshape(1, ho))


def _tc_final(z3, u3, dinv, c_out):
    n, w = u3.shape
    nb = n // ROWB

    def body(z_ref, u_ref, dv_ref, o_ref):
        p = (z_ref[...] + u_ref[...]) * dv_ref[...]
        col = lax.broadcasted_iota(jnp.int32, p.shape, 1)
        valid = col < c_out
        m = jnp.max(jnp.where(valid, p, -jnp.inf), axis=1, keepdims=True)
        e = jnp.where(valid, jnp.exp(p - m), 0.0)
        lse = jnp.log(jnp.sum(e, axis=1, keepdims=True)) + m
        o_ref[...] = (p - lse)[:, :c_out]

    return pl.pallas_call(
        body,
        grid=(nb,),
        in_specs=[
            pl.BlockSpec((ROWB, w), lambda i: (i, 0)),
            pl.BlockSpec((ROWB, w), lambda i: (i, 0)),
            pl.BlockSpec((ROWB, 1), lambda i: (i, 0)),
        ],
        out_specs=pl.BlockSpec((ROWB, c_out), lambda i: (i, 0)),
        out_shape=jax.ShapeDtypeStruct((n, c_out), jnp.float32),
    )(z3, u3, dinv)


def kernel(x, edge_index, W1, b1, g1, be1, W2, b2, g2, be2, W3, b3):
    n = x.shape[0]
    e = edge_index.shape[1]
    c_out = W3.shape[1]

    ep = -(-e // (CH * NC * NS)) * (CH * NC * NS)
    src = edge_index[0]
    dst = edge_index[1]
    if ep != e:
        src = jnp.concatenate([src, jnp.zeros((ep - e,), jnp.int32)])
        dst = jnp.concatenate([dst, jnp.full((ep - e,), -1, jnp.int32)])
    src2d = src.reshape(-1, CH)
    dst2d = dst.reshape(-1, CH)

    wpad = 256
    w3p = jnp.pad(W3, ((0, 0), (0, wpad - c_out)))
    b3p = jnp.pad(b3, (0, wpad - c_out))

    npad = -(-n // (NS * 128)) * (NS * 128)

    degp = _sc_degree(dst2d, npad)
    u1, dinv = _tc_lin1(x, W1, b1, degp)
    z1 = _sc_agg_cols(src2d, dst2d, u1, npad)
    z1 = _sc_agg_cols(src2d, dst2d, u1, npad)
    st1 = _tc_stats(z1, u1, dinv)
    u2 = _tc_layer(z1, u1, dinv, st1, g1, be1, W2, b2, n)
    z2 = _sc_agg_cols(src2d, dst2d, u2, npad)
    st2 = _tc_stats(z2, u2, dinv)
    u3 = _tc_layer(z2, u2, dinv, st2, g2, be2, w3p, b3p, n)
    z3 = _sc_agg_cols(src2d, dst2d, u3, npad, passes=4)
    return _tc_final(z3, u3, dinv, c_out)


# speedup vs baseline: 1.0003x; 1.0003x over previous
import functools

import jax
import jax.numpy as jnp
from jax import lax
from jax.experimental import pallas as pl
from jax.experimental.pallas import tpu as pltpu
from jax.experimental.pallas import tpu_sc as plsc

EPS = 1e-5
ROWB = 1000
NC = 2
NS = 16
CH = 128


def _mesh():
    return plsc.VectorSubcoreMesh(
        core_axis_name="c", subcore_axis_name="s", num_cores=NC, num_subcores=NS
    )


def _zero_fill(buf, rows, width):
    per_row = width // 16

    def body(k, _):
        buf[k // per_row, pl.ds((k % per_row) * 16, 16)] = jnp.zeros((16,), jnp.float32)
        return 0

    lax.fori_loop(0, rows * per_row, body, 0)


def _sc_degree(dst2d, n):
    r = dst2d.shape[0]
    rw = r // (NC * NS)
    rpt = n // NS
    zbr = rpt // 5

    @functools.partial(
        pl.kernel,
        out_type=jax.ShapeDtypeStruct((NC, n, 16), jnp.float32),
        mesh=_mesh(),
        scratch_types=[
            pltpu.VMEM((rw, CH), jnp.int32),
            pltpu.VMEM((CH, 16), jnp.float32),
            pltpu.VMEM((zbr, 16), jnp.float32),
            pltpu.VMEM_SHARED((n, 16), jnp.float32),
        ],
    )
    def k(dst_hbm, out_hbm, dstv, ones_v, zbuf, acc):
        c = lax.axis_index("c")
        s = lax.axis_index("s")
        wid = s * NC + c

        def init_body(i, _):
            ones_v[i, :] = jnp.ones((16,), jnp.float32)
            return 0

        lax.fori_loop(0, CH, init_body, 0)
        _zero_fill(zbuf, zbr, 16)
        for t in range(rpt // zbr):
            pltpu.sync_copy(zbuf, acc.at[pl.ds(s * rpt + t * zbr, zbr)])
        pltpu.sync_copy(dst_hbm.at[pl.ds(wid * rw, rw)], dstv)
        plsc.subcore_barrier()

        def body(j, _):
            pltpu.sync_copy(
                ones_v,
                acc.at[plsc.Indices(dstv.at[j], ignored_value=-1)],
                add=True,
            )
            return 0

        lax.fori_loop(0, rw, body, 0)
        plsc.subcore_barrier()
        pltpu.sync_copy(acc.at[pl.ds(s * rpt, rpt)], out_hbm.at[c, pl.ds(s * rpt, rpt)])

    return k(dst2d)


def _sc_agg_cols(src2d, dst2d, u, n, passes=2):
    r = src2d.shape[0]
    rs_ = r // NS
    w = u.shape[1]
    hw = w // NC
    half = n // passes
    rpt = half // NS
    zbr = rpt // 5

    @functools.partial(
        pl.kernel,
        out_type=jax.ShapeDtypeStruct((n, w), jnp.float32),
        mesh=_mesh(),
        scratch_types=[
            pltpu.VMEM((rs_, CH), jnp.int32),
            pltpu.VMEM((rs_, CH), jnp.int32),
            pltpu.VMEM((1, CH), jnp.int32),
            pltpu.VMEM((1, CH), jnp.int32),
            pltpu.VMEM((1, CH), jnp.int32),
            pltpu.VMEM((1, CH), jnp.int32),
            pltpu.VMEM((CH, hw), jnp.float32),
            pltpu.VMEM((CH, hw), jnp.float32),
            pltpu.VMEM((zbr, hw), jnp.float32),
            pltpu.VMEM_SHARED((half, hw), jnp.float32),
            pltpu.SemaphoreType.DMA,
            pltpu.SemaphoreType.DMA,
        ],
    )
    def k(src_hbm, dst_hbm, u_hbm, z_hbm, srcv, dstv, msa, mda, msb, mdb,
          g0, g1, zbuf, acc, sem0, sem1):
        c = lax.axis_index("c")
        s = lax.axis_index("s")
        cbase = c * hw
        _zero_fill(zbuf, zbr, hw)
        pltpu.sync_copy(src_hbm.at[pl.ds(s * rs_, rs_)], srcv)
        pltpu.sync_copy(dst_hbm.at[pl.ds(s * rs_, rs_)], dstv)

        def gather(ms, gb, sem):
            return pltpu.async_copy(
                u_hbm.at[ms.at[0], pl.ds(cbase, hw)], gb, sem
            )

        def gather_wait(ms, gb, sem):
            pltpu.make_async_copy(
                u_hbm.at[ms.at[0], pl.ds(cbase, hw)], gb, sem
            ).wait()

        def scatter(md, gb):
            pltpu.sync_copy(
                gb,
                acc.at[plsc.Indices(md.at[0], ignored_value=-1)],
                add=True,
            )

        for p in range(passes):
            base = p * half
            for t in range(rpt // zbr):
                pltpu.sync_copy(zbuf, acc.at[pl.ds(s * rpt + t * zbr, zbr)])

            def mask_into(j, ms, md):
                for kk in range(CH // 16):
                    sl = pl.ds(kk * 16, 16)
                    s16 = srcv[j, sl]
                    d16 = dstv[j, sl]
                    inr = (d16 >= base) & (d16 < base + half)
                    ms[0, sl] = jnp.where(inr, s16, 0)
                    md[0, sl] = jnp.where(inr, d16 - base, -1)

            plsc.subcore_barrier()
            mask_into(0, msa, mda)
            gather(msa, g0, sem0)

            def body(j, _):
                @pl.when(lax.rem(j, 2) == 0)
                def _():
                    @pl.when(j + 1 < rs_)
                    def _():
                        mask_into(j + 1, msb, mdb)
                        gather(msb, g1, sem1)

                    gather_wait(msa, g0, sem0)
                    scatter(mda, g0)

                @pl.when(lax.rem(j, 2) == 1)
                def _():
                    @pl.when(j + 1 < rs_)
                    def _():
                        mask_into(j + 1, msa, mda)
                        gather(msa, g0, sem0)

                    gather_wait(msb, g1, sem1)
                    scatter(mdb, g1)

                return 0

            lax.fori_loop(0, rs_, body, 0)
            plsc.subcore_barrier()
            pltpu.sync_copy(
                acc.at[pl.ds(s * rpt, rpt)],
                z_hbm.at[pl.ds(base + s * rpt, rpt), pl.ds(cbase, hw)],
            )
            if p + 1 < passes:
                plsc.subcore_barrier()

    return k(src2d, dst2d, u)


def _tc_lin1(x, w1, b1, degp):
    n, d = x.shape
    h = w1.shape[1]
    nb = n // ROWB

    def body(x_ref, w_ref, b_ref, deg_ref, u_ref, dinv_ref):
        deg = deg_ref[0, :, 0] + deg_ref[1, :, 0] + 1.0
        dv = lax.rsqrt(deg)
        dinv_ref[...] = dv[:, None]
        hh = jnp.dot(x_ref[...], w_ref[...], preferred_element_type=jnp.float32)
        u_ref[...] = (hh + b_ref[...]) * dv[:, None]

    return pl.pallas_call(
        body,
        grid=(nb,),
        in_specs=[
            pl.BlockSpec((ROWB, d), lambda i: (i, 0)),
            pl.BlockSpec((d, h), lambda i: (0, 0)),
            pl.BlockSpec((1, h), lambda i: (0, 0)),
            pl.BlockSpec((NC, ROWB, 16), lambda i: (0, i, 0)),
        ],
        out_specs=[
            pl.BlockSpec((ROWB, h), lambda i: (i, 0)),
            pl.BlockSpec((ROWB, 1), lambda i: (i, 0)),
        ],
        out_shape=[
            jax.ShapeDtypeStruct((n, h), jnp.float32),
            jax.ShapeDtypeStruct((n, 1), jnp.float32),
        ],
    )(x, w1, b1.reshape(1, h), degp)


def _tc_stats(z, u, dinv):
    n, h = u.shape
    nb = n // ROWB

    def body(z_ref, u_ref, dv_ref, s_ref):
        i = pl.program_id(0)
        a = (z_ref[...] + u_ref[...]) * dv_ref[...]

        @pl.when(i == 0)
        def _():
            s_ref[...] = jnp.zeros_like(s_ref)

        s_ref[0:1, :] += jnp.sum(a, axis=0, keepdims=True)
        s_ref[1:2, :] += jnp.sum(a * a, axis=0, keepdims=True)

    return pl.pallas_call(
        body,
        grid=(nb,),
        in_specs=[
            pl.BlockSpec((ROWB, h), lambda i: (i, 0)),
            pl.BlockSpec((ROWB, h), lambda i: (i, 0)),
            pl.BlockSpec((ROWB, 1), lambda i: (i, 0)),
        ],
        out_specs=pl.BlockSpec((8, h), lambda i: (0, 0)),
        out_shape=jax.ShapeDtypeStruct((8, h), jnp.float32),
    )(z, u, dinv)


def _tc_layer(z, u, dinv, st, g, be, w, b, n_total):
    n, h = u.shape
    ho = w.shape[1]
    nb = n // ROWB

    def body(z_ref, u_ref, dv_ref, st_ref, g_ref, be_ref, w_ref, b_ref, o_ref):
        a = (z_ref[...] + u_ref[...]) * dv_ref[...]
        mu = st_ref[0:1, :] * (1.0 / n_total)
        ex2 = st_ref[1:2, :] * (1.0 / n_total)
        var = ex2 - mu * mu
        sc = g_ref[...] * lax.rsqrt(var + EPS)
        tt = be_ref[...] - mu * sc
        hn = jnp.maximum(a * sc + tt, 0.0)
        hh = jnp.dot(hn, w_ref[...], preferred_element_type=jnp.float32)
        o_ref[...] = (hh + b_ref[...]) * dv_ref[...]

    return pl.pallas_call(
        body,
        grid=(nb,),
        in_specs=[
            pl.BlockSpec((ROWB, h), lambda i: (i, 0)),
            pl.BlockSpec((ROWB, h), lambda i: (i, 0)),
            pl.BlockSpec((ROWB, 1), lambda i: (i, 0)),
            pl.BlockSpec((8, h), lambda i: (0, 0)),
            pl.BlockSpec((1, h), lambda i: (0, 0)),
            pl.BlockSpec((1, h), lambda i: (0, 0)),
            pl.BlockSpec((h, ho), lambda i: (0, 0)),
            pl.BlockSpec((1, ho), lambda i: (0, 0)),
        ],
        out_specs=pl.BlockSpec((ROWB, ho), lambda i: (i, 0)),
        out_shape=jax.ShapeDtypeStruct((n, ho), jnp.float32),
    )(z, u, dinv, st, g.reshape(1, h), be.reshape(1, h), w, b.reshape(1, ho))


def _tc_final(z3, u3, dinv, c_out):
    n, w = u3.shape
    nb = n // ROWB

    def body(z_ref, u_ref, dv_ref, o_ref):
        p = (z_ref[...] + u_ref[...]) * dv_ref[...]
        col = lax.broadcasted_iota(jnp.int32, p.shape, 1)
        valid = col < c_out
        m = jnp.max(jnp.where(valid, p, -jnp.inf), axis=1, keepdims=True)
        e = jnp.where(valid, jnp.exp(p - m), 0.0)
        lse = jnp.log(jnp.sum(e, axis=1, keepdims=True)) + m
        o_ref[...] = (p - lse)[:, :c_out]

    return pl.pallas_call(
        body,
        grid=(nb,),
        in_specs=[
            pl.BlockSpec((ROWB, w), lambda i: (i, 0)),
            pl.BlockSpec((ROWB, w), lambda i: (i, 0)),
            pl.BlockSpec((ROWB, 1), lambda i: (i, 0)),
        ],
        out_specs=pl.BlockSpec((ROWB, c_out), lambda i: (i, 0)),
        out_shape=jax.ShapeDtypeStruct((n, c_out), jnp.float32),
    )(z3, u3, dinv)


def kernel(x, edge_index, W1, b1, g1, be1, W2, b2, g2, be2, W3, b3):
    n = x.shape[0]
    e = edge_index.shape[1]
    c_out = W3.shape[1]

    ep = -(-e // (CH * NC * NS)) * (CH * NC * NS)
    src = edge_index[0]
    dst = edge_index[1]
    if ep != e:
        src = jnp.concatenate([src, jnp.zeros((ep - e,), jnp.int32)])
        dst = jnp.concatenate([dst, jnp.full((ep - e,), -1, jnp.int32)])
    src2d = src.reshape(-1, CH)
    dst2d = dst.reshape(-1, CH)

    wpad = 256
    w3p = jnp.pad(W3, ((0, 0), (0, wpad - c_out)))
    b3p = jnp.pad(b3, (0, wpad - c_out))

    npad = -(-n // (NS * 128)) * (NS * 128)

    degp = _sc_degree(dst2d, npad)
    u1, dinv = _tc_lin1(x, W1, b1, degp)
    z1 = _sc_agg_cols(src2d, dst2d, u1, npad)
    z1 = _sc_agg_cols(src2d, dst2d, u1, npad)
    st1 = _tc_stats(z1, u1, dinv)
    u2 = _tc_layer(z1, u1, dinv, st1, g1, be1, W2, b2, n)
    z2 = _sc_agg_cols(src2d, dst2d, u2, npad)
    st2 = _tc_stats(z2, u2, dinv)
    u3 = _tc_layer(z2, u2, dinv, st2, g2, be2, w3p, b3p, n)
    z3 = _sc_agg_cols(src2d, dst2d, u3, npad, passes=4)
    return _tc_final(z3, u3, dinv, c_out)
